# baseline (speedup 1.0000x reference)
import functools
import math

import jax
import jax.numpy as jnp
import numpy as np
from jax import lax
from jax.experimental import pallas as pl
from jax.experimental.pallas import tpu as pltpu
from jax.experimental.pallas import tpu_sc as plsc

B = 16384
D_E = 32
D_M = 128
CH = 128
BLK = 2048
SCALE = np.float32(np.sqrt(float(D_M)))


def _sc_gather(f0, f1, f2, emb0, emb1, emb2):
    mesh = plsc.VectorSubcoreMesh(core_axis_name="c", subcore_axis_name="s")
    nw = mesh.num_cores * mesh.num_subcores
    bpw = B // nw
    nch = bpw // CH
    fr = [f.reshape(nw, nch, CH) for f in (f0, f1, f2)]

    @functools.partial(
        pl.kernel,
        out_type=[jax.ShapeDtypeStruct((B, D_E), jnp.float32) for _ in range(3)],
        mesh=mesh,
        scratch_types=(
            [pltpu.VMEM((nch, CH), jnp.int32) for _ in range(3)]
            + [pltpu.VMEM((bpw, D_E), jnp.float32) for _ in range(3)]
            + [pltpu.SemaphoreType.DMA]
        ),
        compiler_params=pltpu.CompilerParams(use_tc_tiling_on_sc=False),
    )
    def gather_kernel(f0h, f1h, f2h, e0h, e1h, e2h, o0h, o1h, o2h,
                      i0v, i1v, i2v, r0v, r1v, r2v, sem):
        wid = lax.axis_index("s") * mesh.num_cores + lax.axis_index("c")
        base = wid * bpw
        fhs = (f0h, f1h, f2h)
        ehs = (e0h, e1h, e2h)
        ohs = (o0h, o1h, o2h)
        ivs = (i0v, i1v, i2v)
        rvs = (r0v, r1v, r2v)
        for t in range(3):
            pltpu.sync_copy(fhs[t].at[wid], ivs[t])
        descs = []
        for t in range(3):
            for j in range(nch):
                descs.append(pltpu.async_copy(
                    ehs[t].at[ivs[t].at[j]], rvs[t].at[pl.ds(j * CH, CH)], sem))
        for dsc in descs:
            dsc.wait()
        for t in range(3):
            pltpu.sync_copy(rvs[t], ohs[t].at[pl.ds(base, bpw)])

    return gather_kernel(*fr, emb0, emb1, emb2)


def _proj_body(x0r, x1r, x2r, f3r, lwtr, lbr, w0r, w1r, w2r, w3r, pbr, outr):
    x3 = f3r[...] * lwtr[...] + lbr[...]
    acc = jnp.dot(x0r[...], w0r[...], preferred_element_type=jnp.float32)
    acc += jnp.dot(x1r[...], w1r[...], preferred_element_type=jnp.float32)
    acc += jnp.dot(x2r[...], w2r[...], preferred_element_type=jnp.float32)
    acc += jnp.dot(x3, w3r[...], preferred_element_type=jnp.float32)
    outr[...] = (acc + pbr[...]) * SCALE


def kernel(f0, f1, f2, f3, emb0, emb1, emb2, lin_w, lin_b, proj_w, proj_b):
    x0, x1, x2 = _sc_gather(f0, f1, f2, emb0, emb1, emb2)

    lin_wT = lin_w.reshape(1, 16)
    lin_b2 = lin_b.reshape(1, 16)
    w0 = proj_w[:, 0:32].T
    w1 = proj_w[:, 32:64].T
    w2 = proj_w[:, 64:96].T
    w3 = proj_w[:, 96:112].T
    pb = proj_b.reshape(1, D_M)

    cst = lambda i: (0, 0)
    out = pl.pallas_call(
        _proj_body,
        grid=(B // BLK,),
        in_specs=[
            pl.BlockSpec((BLK, D_E), lambda i: (i, 0)),
            pl.BlockSpec((BLK, D_E), lambda i: (i, 0)),
            pl.BlockSpec((BLK, D_E), lambda i: (i, 0)),
            pl.BlockSpec((BLK, 1), lambda i: (i, 0)),
            pl.BlockSpec((1, 16), cst),
            pl.BlockSpec((1, 16), cst),
            pl.BlockSpec((D_E, D_M), cst),
            pl.BlockSpec((D_E, D_M), cst),
            pl.BlockSpec((D_E, D_M), cst),
            pl.BlockSpec((16, D_M), cst),
            pl.BlockSpec((1, D_M), cst),
        ],
        out_specs=pl.BlockSpec((BLK, D_M), lambda i: (i, 0)),
        out_shape=jax.ShapeDtypeStruct((B, D_M), jnp.float32),
    )(x0, x1, x2, f3, lin_wT, lin_b2, w0, w1, w2, w3, pb)
    return out
